# Initial kernel scaffold; baseline (speedup 1.0000x reference)
import jax
import jax.numpy as jnp
from jax.experimental import pallas as pl


def kernel(x, edge_index, batch, emb_table, W1, a_src1, a_dst1, b1, W2, a_src2, a_dst2, b2, lin_w, lin_b):
    raise NotImplementedError("write your pallas kernel here")


# speedup vs baseline: 1.0010x; 1.0010x over previous
import functools

import jax
import jax.numpy as jnp
from jax.experimental import pallas as pl
from jax.experimental.pallas import tpu as pltpu

N = 50000
E = 800000
VOCAB = 100000
EMB = 64
HID = 128
HEADS = 4
C = HID // HEADS
NCLASS = 2
NGRAPH = 64


def _mm_kernel(x_ref, w_ref, o_ref):
    o_ref[...] = jnp.dot(x_ref[...], w_ref[...],
                         preferred_element_type=jnp.float32)


def _matmul(x, w, block_m=2000):
    m, k = x.shape
    _, n = w.shape
    grid = (m // block_m,)
    return pl.pallas_call(
        _mm_kernel,
        grid=grid,
        in_specs=[
            pl.BlockSpec((block_m, k), lambda i: (i, 0)),
            pl.BlockSpec((k, n), lambda i: (0, 0)),
        ],
        out_specs=pl.BlockSpec((block_m, n), lambda i: (i, 0)),
        out_shape=jax.ShapeDtypeStruct((m, n), jnp.float32),
    )(x, w)


def _gat(h, src, dst, W, a_s, a_d, b):
    n = h.shape[0]
    hp = _matmul(h, W).reshape(n, HEADS, C)
    alpha_s = (hp * a_s[None, :, :]).sum(-1)
    alpha_d = (hp * a_d[None, :, :]).sum(-1)
    alpha = jax.nn.leaky_relu(alpha_s[src] + alpha_d[dst], 0.2)
    amax = jax.ops.segment_max(alpha, dst, num_segments=n)
    amax = jnp.where(jnp.isfinite(amax), amax, 0.0)
    ex = jnp.exp(alpha - amax[dst])
    denom = jax.ops.segment_sum(ex, dst, num_segments=n)
    coef = ex / (denom[dst] + 1e-16)
    out = jax.ops.segment_sum(hp[src] * coef[:, :, None], dst, num_segments=n)
    return out.reshape(n, HID) + b


def kernel(x, edge_index, batch, emb_table, W1, a_src1, a_dst1, b1,
           W2, a_src2, a_dst2, b2, lin_w, lin_b):
    loop = jnp.arange(N)
    src = jnp.concatenate([edge_index[0], loop])
    dst = jnp.concatenate([edge_index[1], loop])
    h = jnp.take(emb_table, x, axis=0)
    h = jax.nn.relu(_gat(h, src, dst, W1, a_src1, a_dst1, b1))
    h = jax.nn.relu(_gat(h, src, dst, W2, a_src2, a_dst2, b2))
    sums = jax.ops.segment_sum(h, batch, num_segments=NGRAPH)
    cnt = jax.ops.segment_sum(jnp.ones((N,), dtype=h.dtype), batch,
                              num_segments=NGRAPH)
    mean = sums / jnp.maximum(cnt, 1.0)[:, None]
    return mean @ lin_w + lin_b


# speedup vs baseline: 32.2660x; 32.2323x over previous
import functools

import jax
import jax.numpy as jnp
from jax import lax
from jax.experimental import pallas as pl
from jax.experimental.pallas import tpu as pltpu
from jax.experimental.pallas import tpu_sc as plsc

_SC_NC = 2
_SC_NS = 16
_NW = _SC_NC * _SC_NS

N = 50000
E = 800000
VOCAB = 100000
EMB = 64
HID = 128
HEADS = 4
C = HID // HEADS
NCLASS = 2
NGRAPH = 64


def _sc_mesh():
    return plsc.VectorSubcoreMesh(core_axis_name="c", subcore_axis_name="s",
                                  num_cores=_SC_NC, num_subcores=_SC_NS)


def _emb_gather(table, idx_pad):
    B = idx_pad.shape[0]
    D = table.shape[1]
    bpw = B // _NW

    @functools.partial(
        pl.kernel,
        out_type=jax.ShapeDtypeStruct((B, D), jnp.float32),
        mesh=_sc_mesh(),
        compiler_params=pltpu.CompilerParams(use_tc_tiling_on_sc=False),
        scratch_types=[
            pltpu.VMEM((bpw,), jnp.int32),
            pltpu.VMEM((bpw, D), jnp.float32),
            pltpu.SemaphoreType.DMA,
        ],
    )
    def k(table_hbm, idx_hbm, out_hbm, idx_v, rows_v, sem):
        wid = lax.axis_index("s") * _SC_NC + lax.axis_index("c")
        base = wid * bpw
        pltpu.sync_copy(idx_hbm.at[pl.ds(base, bpw)], idx_v)
        pltpu.async_copy(table_hbm.at[idx_v], rows_v, sem).wait()
        pltpu.sync_copy(rows_v, out_hbm.at[pl.ds(base, bpw)])

    return k(table, idx_pad)


E_PAD = 819200
NPAD = 50048
_EPT = E_PAD // _NW
_KCH = 5120
_NCHUNK = _EPT // _KCH
_G = 256
_KB = 2560
_NCB = _EPT // _KB
_NWIN = _KB // _G


_GDN = lax.GatherDimensionNumbers(offset_dims=(), collapsed_slice_dims=(0,),
                                  start_index_map=(0,))


def _vdyn(v, idx16):
    return lax.gather(v, idx16[:, None], _GDN, (1,),
                      mode=lax.GatherScatterMode.PROMISE_IN_BOUNDS)


def _edge_phase_a(src, dst, ast, adt, zf):
    @functools.partial(
        pl.kernel,
        out_type=(jax.ShapeDtypeStruct((2, NPAD * 4), jnp.float32),
                  jax.ShapeDtypeStruct((E_PAD * 4,), jnp.float32)),
        mesh=_sc_mesh(),
        compiler_params=pltpu.CompilerParams(use_tc_tiling_on_sc=False),
        scratch_types=[
            pltpu.VMEM((_KCH,), jnp.int32),
            pltpu.VMEM((_KCH,), jnp.int32),
            pltpu.VMEM((_KCH * 4,), jnp.int32),
            pltpu.VMEM((_KCH * 4,), jnp.int32),
            pltpu.VMEM((_KCH * 4,), jnp.float32),
            pltpu.VMEM((_KCH * 4,), jnp.float32),
            pltpu.VMEM((_KCH * 4,), jnp.float32),
            pltpu.VMEM_SHARED((NPAD * 4,), jnp.float32),
            pltpu.SemaphoreType.DMA,
            pltpu.SemaphoreType.DMA,
        ],
    )
    def k(src_h, dst_h, ast_h, adt_h, zf_h, den_o, ex_o,
          srcv, dstv, idxs4, idxd4, asb, adb, exb, den_sh, sem1, sem2):
        cid = lax.axis_index("c")
        sid = lax.axis_index("s")
        wid = sid * _SC_NC + cid
        ebase = wid * _EPT
        rows = NPAD // _SC_NS
        r0 = sid * rows

        pltpu.sync_copy(zf_h.at[pl.ds(r0 * 4, rows * 4)],
                        den_sh.at[pl.ds(r0 * 4, rows * 4)])
        plsc.subcore_barrier()

        iotaq = lax.shift_right_logical(lax.iota(jnp.int32, 16), 2)
        iotam = lax.bitwise_and(lax.iota(jnp.int32, 16), 3)

        def chunkA(c, _):
            off = ebase + c * _KCH
            pltpu.sync_copy(src_h.at[pl.ds(off, _KCH)], srcv)
            pltpu.sync_copy(dst_h.at[pl.ds(off, _KCH)], dstv)

            def expand(t, _):
                sv = srcv[pl.ds(16 * t, 16)]
                dv = dstv[pl.ds(16 * t, 16)]
                for j in range(4):
                    sel = iotaq + 4 * j
                    idxs4[pl.ds(64 * t + 16 * j, 16)] = (
                        _vdyn(sv, sel) * 4 + iotam)
                    idxd4[pl.ds(64 * t + 16 * j, 16)] = (
                        _vdyn(dv, sel) * 4 + iotam)
                return 0

            lax.fori_loop(0, _KCH // 16, expand, 0)
            cp1 = pltpu.async_copy(ast_h.at[idxs4], asb, sem1)
            cp2 = pltpu.async_copy(adt_h.at[idxd4], adb, sem2)
            cp1.wait()
            cp2.wait()

            def grp(g, _):
                a = asb[pl.ds(16 * g, 16)] + adb[pl.ds(16 * g, 16)]
                al = jnp.maximum(a, 0.2 * a)
                e = jnp.exp(al)
                eid = iotaq + (off + 4 * g)
                e = jnp.where(eid < E, e, 0.0)
                exb[pl.ds(16 * g, 16)] = e
                return 0

            lax.fori_loop(0, _KCH // 4, grp, 0)
            pltpu.sync_copy(exb, ex_o.at[pl.ds(off * 4, _KCH * 4)])
            pltpu.sync_copy(exb, den_sh.at[idxd4], add=True)
            return 0

        lax.fori_loop(0, _NCHUNK, chunkA, 0)
        plsc.subcore_barrier()
        pltpu.sync_copy(den_sh.at[pl.ds(r0 * 4, rows * 4)],
                        den_o.at[cid, pl.ds(r0 * 4, rows * 4)])

    return k(src, dst, ast, adt, zf)


def _edge_phase_b(src, dst, ex, hps, z32):
    @functools.partial(
        pl.kernel,
        out_type=jax.ShapeDtypeStruct((2, 4, NPAD, 32), jnp.float32),
        mesh=_sc_mesh(),
        compiler_params=pltpu.CompilerParams(use_tc_tiling_on_sc=False),
        scratch_types=[
            pltpu.VMEM((_KB,), jnp.int32),
            pltpu.VMEM((_KB,), jnp.int32),
            pltpu.VMEM((_KB * 4,), jnp.float32),
            pltpu.VMEM((_G, 32), jnp.float32),
            pltpu.VMEM((_G,), jnp.int32),
            pltpu.VMEM_SHARED((NPAD, 32), jnp.float32),
            pltpu.SemaphoreType.DMA,
        ],
    )
    def k(src_h, dst_h, ex_h, hp0, hp1, hp2, hp3, z32_h, acc_o,
          srcv, dstv, exb, hpwin, idxwin, acc_sh, sem1):
        cid = lax.axis_index("c")
        sid = lax.axis_index("s")
        wid = sid * _SC_NC + cid
        ebase = wid * _EPT
        rows = NPAD // _SC_NS
        r0 = sid * rows

        for h in range(4):
            hp_h = (hp0, hp1, hp2, hp3)[h]
            pltpu.sync_copy(z32_h.at[pl.ds(r0, rows)],
                            acc_sh.at[pl.ds(r0, rows)])
            plsc.subcore_barrier()

            def chunkB(c, _):
                off = ebase + c * _KB
                pltpu.sync_copy(src_h.at[pl.ds(off, _KB)], srcv)
                pltpu.sync_copy(dst_h.at[pl.ds(off, _KB)], dstv)
                pltpu.sync_copy(ex_h.at[pl.ds(off * 4, _KB * 4)], exb)

                def win(w, _):
                    def cpidx(t, _):
                        idxwin[pl.ds(16 * t, 16)] = (
                            dstv[pl.ds(w * _G + 16 * t, 16)])
                        return 0

                    lax.fori_loop(0, _G // 16, cpidx, 0)
                    pltpu.async_copy(
                        hp_h.at[srcv.at[pl.ds(w * _G, _G)]], hpwin,
                        sem1).wait()

                    def grp(g, _):
                        e16 = exb[pl.ds((w * _G + 4 * g) * 4, 16)]
                        for i in range(4):
                            b = _vdyn(e16, jnp.full((16,), 4 * i + h,
                                                    jnp.int32))
                            e = 4 * g + i
                            hpwin[e, pl.ds(0, 16)] = (
                                hpwin[e, pl.ds(0, 16)] * b)
                            hpwin[e, pl.ds(16, 16)] = (
                                hpwin[e, pl.ds(16, 16)] * b)
                        return 0

                    lax.fori_loop(0, _G // 4, grp, 0)
                    pltpu.sync_copy(hpwin, acc_sh.at[idxwin], add=True)
                    return 0

                lax.fori_loop(0, _NWIN, win, 0)
                return 0

            lax.fori_loop(0, _NCB, chunkB, 0)
            plsc.subcore_barrier()
            pltpu.sync_copy(acc_sh.at[pl.ds(r0, rows)],
                            acc_o.at[cid, h, pl.ds(r0, rows)])
            plsc.subcore_barrier()

    return k(src, dst, ex, hps[0], hps[1], hps[2], hps[3], z32)


def _mm_kernel(x_ref, w_ref, o_ref):
    o_ref[...] = jnp.dot(x_ref[...], w_ref[...],
                         preferred_element_type=jnp.float32)


def _matmul(x, w, block_m=2000):
    m, k = x.shape
    _, n = w.shape
    grid = (m // block_m,)
    return pl.pallas_call(
        _mm_kernel,
        grid=grid,
        in_specs=[
            pl.BlockSpec((block_m, k), lambda i: (i, 0)),
            pl.BlockSpec((k, n), lambda i: (0, 0)),
        ],
        out_specs=pl.BlockSpec((block_m, n), lambda i: (i, 0)),
        out_shape=jax.ShapeDtypeStruct((m, n), jnp.float32),
    )(x, w)


def _pool_kernel(pt_ref, h_ref, o_ref):
    @pl.when(pl.program_id(0) == 0)
    def _():
        o_ref[...] = jnp.zeros_like(o_ref)

    o_ref[...] += jnp.dot(pt_ref[...], h_ref[...],
                          preferred_element_type=jnp.float32)


def _poolmm(pt, h, block_k=6272):
    k = pt.shape[1]
    grid = (k // block_k,)
    return pl.pallas_call(
        _pool_kernel,
        grid=grid,
        in_specs=[
            pl.BlockSpec((NGRAPH, block_k), lambda i: (0, i)),
            pl.BlockSpec((block_k, HID), lambda i: (i, 0)),
        ],
        out_specs=pl.BlockSpec((NGRAPH, HID), lambda i: (0, 0)),
        out_shape=jax.ShapeDtypeStruct((NGRAPH, HID), jnp.float32),
    )(pt, h)


def _gat_sc(h, srcp, dstp, W, a_s, a_d, b, zf, z32):
    hp = _matmul(h, W)
    hsel = jnp.arange(HID) // C
    onehot_h = (hsel[:, None] == jnp.arange(HEADS)[None, :]).astype(jnp.float32)
    Amat = jnp.zeros((HID, HID), jnp.float32)
    Amat = Amat.at[:, :HEADS].set(onehot_h * a_s.reshape(HID)[:, None])
    Amat = Amat.at[:, HEADS:2 * HEADS].set(
        onehot_h * a_d.reshape(HID)[:, None])
    al = _matmul(hp, Amat)
    alpha_s = al[:, :HEADS]
    alpha_d = al[:, HEADS:2 * HEADS]

    pad_n = NPAD - N
    ast = jnp.pad(alpha_s, ((0, pad_n), (0, 0))).ravel()
    adt = jnp.pad(alpha_d, ((0, pad_n), (0, 0))).ravel()
    den_p, ex = _edge_phase_a(srcp, dstp, ast, adt, zf)
    den = (den_p[0] + den_p[1]).reshape(NPAD, HEADS)[:N]

    aself = alpha_s + alpha_d
    exself = jnp.exp(jnp.maximum(aself, 0.2 * aself))
    denom = den + exself

    hp_pad = jnp.pad(hp, ((0, pad_n), (0, 0)))
    hps = tuple(hp_pad[:, i * C:(i + 1) * C] for i in range(HEADS))
    acc_p = _edge_phase_b(srcp, dstp, ex, hps, z32)
    acc = (acc_p[0] + acc_p[1]).transpose(1, 0, 2)[:N]
    numer = acc + exself[:, :, None] * hp.reshape(N, HEADS, C)
    out = numer / (denom[:, :, None] + 1e-16)
    return out.reshape(N, HID) + b


def kernel(x, edge_index, batch, emb_table, W1, a_src1, a_dst1, b1,
           W2, a_src2, a_dst2, b2, lin_w, lin_b):
    epad = E_PAD - E
    srcp = jnp.concatenate([edge_index[0].astype(jnp.int32),
                            jnp.zeros((epad,), jnp.int32)])
    dstp = jnp.concatenate([edge_index[1].astype(jnp.int32),
                            jnp.zeros((epad,), jnp.int32)])
    zf = jnp.zeros((NPAD * HEADS,), jnp.float32)
    z32 = jnp.zeros((NPAD, C), jnp.float32)

    npad = (-N) % (8 * _NW)
    x_pad = jnp.concatenate([x.astype(jnp.int32),
                             jnp.arange(npad, dtype=jnp.int32)])
    h = _emb_gather(emb_table, x_pad)[:N]
    h = jax.nn.relu(_gat_sc(h, srcp, dstp, W1, a_src1, a_dst1, b1, zf, z32))
    h = jax.nn.relu(_gat_sc(h, srcp, dstp, W2, a_src2, a_dst2, b2, zf, z32))

    pool_pad = 50176
    pt = (jnp.arange(NGRAPH)[:, None] == batch[None, :]).astype(jnp.float32)
    pt_pad = jnp.pad(pt, ((0, 0), (0, pool_pad - N)))
    h_pad = jnp.pad(h, ((0, pool_pad - N), (0, 0)))
    sums = _poolmm(pt_pad, h_pad)
    cnt = jnp.sum(pt, axis=1)
    mean = sums / jnp.maximum(cnt, 1.0)[:, None]
    lin_w_pad = jnp.pad(lin_w, ((0, 0), (0, HID - NCLASS)))
    out = _matmul(mean, lin_w_pad, block_m=NGRAPH)[:, :NCLASS]
    return out + lin_b


# speedup vs baseline: 36.9208x; 1.1443x over previous
import functools

import jax
import jax.numpy as jnp
from jax import lax
from jax.experimental import pallas as pl
from jax.experimental.pallas import tpu as pltpu
from jax.experimental.pallas import tpu_sc as plsc

_SC_NC = 2
_SC_NS = 16
_NW = _SC_NC * _SC_NS

N = 50000
E = 800000
VOCAB = 100000
EMB = 64
HID = 128
HEADS = 4
C = HID // HEADS
NCLASS = 2
NGRAPH = 64


def _sc_mesh():
    return plsc.VectorSubcoreMesh(core_axis_name="c", subcore_axis_name="s",
                                  num_cores=_SC_NC, num_subcores=_SC_NS)


def _emb_gather(table, idx_pad):
    B = idx_pad.shape[0]
    D = table.shape[1]
    bpw = B // _NW

    @functools.partial(
        pl.kernel,
        out_type=jax.ShapeDtypeStruct((B, D), jnp.float32),
        mesh=_sc_mesh(),
        compiler_params=pltpu.CompilerParams(use_tc_tiling_on_sc=False),
        scratch_types=[
            pltpu.VMEM((bpw,), jnp.int32),
            pltpu.VMEM((bpw, D), jnp.float32),
            pltpu.SemaphoreType.DMA,
        ],
    )
    def k(table_hbm, idx_hbm, out_hbm, idx_v, rows_v, sem):
        wid = lax.axis_index("s") * _SC_NC + lax.axis_index("c")
        base = wid * bpw
        pltpu.sync_copy(idx_hbm.at[pl.ds(base, bpw)], idx_v)
        pltpu.async_copy(table_hbm.at[idx_v], rows_v, sem).wait()
        pltpu.sync_copy(rows_v, out_hbm.at[pl.ds(base, bpw)])

    return k(table, idx_pad)


E_PAD = 819200
NPAD = 50048
_EPT = E_PAD // _NW
_KCH = 5120
_NCHUNK = _EPT // _KCH
_G = 128
_KB = 2560
_NCB = _EPT // _KB
_NWIN = _KB // _G


_GDN = lax.GatherDimensionNumbers(offset_dims=(), collapsed_slice_dims=(0,),
                                  start_index_map=(0,))


def _vdyn(v, idx16):
    return lax.gather(v, idx16[:, None], _GDN, (1,),
                      mode=lax.GatherScatterMode.PROMISE_IN_BOUNDS)


def _edge_phase_a(src, dst, ast, adt, zf):
    @functools.partial(
        pl.kernel,
        out_type=(jax.ShapeDtypeStruct((2, NPAD * 4), jnp.float32),
                  jax.ShapeDtypeStruct((E_PAD * 4,), jnp.float32)),
        mesh=_sc_mesh(),
        compiler_params=pltpu.CompilerParams(use_tc_tiling_on_sc=False),
        scratch_types=[
            pltpu.VMEM((_KCH,), jnp.int32),
            pltpu.VMEM((_KCH,), jnp.int32),
            pltpu.VMEM((_KCH * 4,), jnp.int32),
            pltpu.VMEM((_KCH * 4,), jnp.int32),
            pltpu.VMEM((_KCH * 4,), jnp.float32),
            pltpu.VMEM((_KCH * 4,), jnp.float32),
            pltpu.VMEM((_KCH * 4,), jnp.float32),
            pltpu.VMEM_SHARED((NPAD * 4,), jnp.float32),
            pltpu.SemaphoreType.DMA,
            pltpu.SemaphoreType.DMA,
        ],
    )
    def k(src_h, dst_h, ast_h, adt_h, zf_h, den_o, ex_o,
          srcv, dstv, idxs4, idxd4, asb, adb, exb, den_sh, sem1, sem2):
        cid = lax.axis_index("c")
        sid = lax.axis_index("s")
        wid = sid * _SC_NC + cid
        ebase = wid * _EPT
        rows = NPAD // _SC_NS
        r0 = sid * rows

        pltpu.sync_copy(zf_h.at[pl.ds(r0 * 4, rows * 4)],
                        den_sh.at[pl.ds(r0 * 4, rows * 4)])
        plsc.subcore_barrier()

        iotaq = lax.shift_right_logical(lax.iota(jnp.int32, 16), 2)
        iotam = lax.bitwise_and(lax.iota(jnp.int32, 16), 3)

        def chunkA(c, _):
            off = ebase + c * _KCH
            pltpu.sync_copy(src_h.at[pl.ds(off, _KCH)], srcv)
            pltpu.sync_copy(dst_h.at[pl.ds(off, _KCH)], dstv)

            def expand(t, _):
                sv = srcv[pl.ds(16 * t, 16)]
                dv = dstv[pl.ds(16 * t, 16)]
                for j in range(4):
                    sel = iotaq + 4 * j
                    idxs4[pl.ds(64 * t + 16 * j, 16)] = (
                        _vdyn(sv, sel) * 4 + iotam)
                    idxd4[pl.ds(64 * t + 16 * j, 16)] = (
                        _vdyn(dv, sel) * 4 + iotam)
                return 0

            lax.fori_loop(0, _KCH // 16, expand, 0)
            cp1 = pltpu.async_copy(ast_h.at[idxs4], asb, sem1)
            cp2 = pltpu.async_copy(adt_h.at[idxd4], adb, sem2)
            cp1.wait()
            cp2.wait()

            def grp(g, _):
                a = asb[pl.ds(16 * g, 16)] + adb[pl.ds(16 * g, 16)]
                al = jnp.maximum(a, 0.2 * a)
                e = jnp.exp(al)
                eid = iotaq + (off + 4 * g)
                e = jnp.where(eid < E, e, 0.0)
                exb[pl.ds(16 * g, 16)] = e
                return 0

            lax.fori_loop(0, _KCH // 4, grp, 0)
            pltpu.sync_copy(exb, ex_o.at[pl.ds(off * 4, _KCH * 4)])
            pltpu.sync_copy(exb, den_sh.at[idxd4], add=True)
            return 0

        lax.fori_loop(0, _NCHUNK, chunkA, 0)
        plsc.subcore_barrier()
        pltpu.sync_copy(den_sh.at[pl.ds(r0 * 4, rows * 4)],
                        den_o.at[cid, pl.ds(r0 * 4, rows * 4)])

    return k(src, dst, ast, adt, zf)


def _edge_phase_b(src, dst, ex, hps, z32):
    @functools.partial(
        pl.kernel,
        out_type=jax.ShapeDtypeStruct((2, 4, NPAD, 32), jnp.float32),
        mesh=_sc_mesh(),
        compiler_params=pltpu.CompilerParams(use_tc_tiling_on_sc=False),
        scratch_types=[
            pltpu.VMEM((_KB,), jnp.int32),
            pltpu.VMEM((_KB,), jnp.int32),
            pltpu.VMEM((_KB * 4,), jnp.float32),
            pltpu.VMEM((_G, 32), jnp.float32),
            pltpu.VMEM((_G, 32), jnp.float32),
            pltpu.VMEM((_G,), jnp.int32),
            pltpu.VMEM_SHARED((NPAD, 32), jnp.float32),
            pltpu.SemaphoreType.DMA,
            pltpu.SemaphoreType.DMA,
        ],
    )
    def k(src_h, dst_h, ex_h, hp0, hp1, hp2, hp3, z32_h, acc_o,
          srcv, dstv, exb, hpwin0, hpwin1, idxwin, acc_sh, sem0, sem1):
        cid = lax.axis_index("c")
        sid = lax.axis_index("s")
        wid = sid * _SC_NC + cid
        ebase = wid * _EPT
        rows = NPAD // _SC_NS
        r0 = sid * rows

        for h in range(4):
            hp_h = (hp0, hp1, hp2, hp3)[h]
            pltpu.sync_copy(z32_h.at[pl.ds(r0, rows)],
                            acc_sh.at[pl.ds(r0, rows)])
            plsc.subcore_barrier()

            def chunkB(c, _):
                off = ebase + c * _KB
                pltpu.sync_copy(src_h.at[pl.ds(off, _KB)], srcv)
                pltpu.sync_copy(dst_h.at[pl.ds(off, _KB)], dstv)
                pltpu.sync_copy(ex_h.at[pl.ds(off * 4, _KB * 4)], exb)

                pltpu.async_copy(hp_h.at[srcv.at[pl.ds(0, _G)]], hpwin0,
                                 sem0)
                pltpu.async_copy(hp_h.at[srcv.at[pl.ds(_G, _G)]], hpwin1,
                                 sem1)

                def win2(w2, _):
                    for par in range(2):
                        w = w2 * 2 + par
                        buf = (hpwin0, hpwin1)[par]
                        sem = (sem0, sem1)[par]

                        def cpidx(t, _):
                            idxwin[pl.ds(16 * t, 16)] = (
                                dstv[pl.ds(w * _G + 16 * t, 16)])
                            return 0

                        lax.fori_loop(0, _G // 16, cpidx, 0)
                        pltpu.make_async_copy(
                            hp_h.at[srcv.at[pl.ds(0, _G)]], buf,
                            sem).wait()

                        def grp(g, _):
                            e16 = exb[pl.ds((w * _G + 4 * g) * 4, 16)]
                            for i in range(4):
                                b = _vdyn(e16, jnp.full((16,), 4 * i + h,
                                                        jnp.int32))
                                e = 4 * g + i
                                buf[e, pl.ds(0, 16)] = (
                                    buf[e, pl.ds(0, 16)] * b)
                                buf[e, pl.ds(16, 16)] = (
                                    buf[e, pl.ds(16, 16)] * b)
                            return 0

                        lax.fori_loop(0, _G // 4, grp, 0)
                        pltpu.sync_copy(buf, acc_sh.at[idxwin], add=True)

                        @pl.when(w + 2 < _NWIN)
                        def _():
                            pltpu.async_copy(
                                hp_h.at[srcv.at[pl.ds((w + 2) * _G, _G)]],
                                buf, sem)
                    return 0

                lax.fori_loop(0, _NWIN // 2, win2, 0)
                return 0

            lax.fori_loop(0, _NCB, chunkB, 0)
            plsc.subcore_barrier()
            pltpu.sync_copy(acc_sh.at[pl.ds(r0, rows)],
                            acc_o.at[cid, h, pl.ds(r0, rows)])
            plsc.subcore_barrier()

    return k(src, dst, ex, hps[0], hps[1], hps[2], hps[3], z32)


def _mm_kernel(x_ref, w_ref, o_ref):
    o_ref[...] = jnp.dot(x_ref[...], w_ref[...],
                         preferred_element_type=jnp.float32)


def _matmul(x, w, block_m=2000):
    m, k = x.shape
    _, n = w.shape
    grid = (m // block_m,)
    return pl.pallas_call(
        _mm_kernel,
        grid=grid,
        in_specs=[
            pl.BlockSpec((block_m, k), lambda i: (i, 0)),
            pl.BlockSpec((k, n), lambda i: (0, 0)),
        ],
        out_specs=pl.BlockSpec((block_m, n), lambda i: (i, 0)),
        out_shape=jax.ShapeDtypeStruct((m, n), jnp.float32),
    )(x, w)


def _pool_kernel(pt_ref, h_ref, o_ref):
    @pl.when(pl.program_id(0) == 0)
    def _():
        o_ref[...] = jnp.zeros_like(o_ref)

    o_ref[...] += jnp.dot(pt_ref[...], h_ref[...],
                          preferred_element_type=jnp.float32)


def _poolmm(pt, h, block_k=6272):
    k = pt.shape[1]
    grid = (k // block_k,)
    return pl.pallas_call(
        _pool_kernel,
        grid=grid,
        in_specs=[
            pl.BlockSpec((NGRAPH, block_k), lambda i: (0, i)),
            pl.BlockSpec((block_k, HID), lambda i: (i, 0)),
        ],
        out_specs=pl.BlockSpec((NGRAPH, HID), lambda i: (0, 0)),
        out_shape=jax.ShapeDtypeStruct((NGRAPH, HID), jnp.float32),
    )(pt, h)


def _gat_sc(h, srcp, dstp, W, a_s, a_d, b, zf, z32):
    hp = _matmul(h, W)
    hsel = jnp.arange(HID) // C
    onehot_h = (hsel[:, None] == jnp.arange(HEADS)[None, :]).astype(jnp.float32)
    Amat = jnp.zeros((HID, HID), jnp.float32)
    Amat = Amat.at[:, :HEADS].set(onehot_h * a_s.reshape(HID)[:, None])
    Amat = Amat.at[:, HEADS:2 * HEADS].set(
        onehot_h * a_d.reshape(HID)[:, None])
    al = _matmul(hp, Amat)
    alpha_s = al[:, :HEADS]
    alpha_d = al[:, HEADS:2 * HEADS]

    pad_n = NPAD - N
    ast = jnp.pad(alpha_s, ((0, pad_n), (0, 0))).ravel()
    adt = jnp.pad(alpha_d, ((0, pad_n), (0, 0))).ravel()
    den_p, ex = _edge_phase_a(srcp, dstp, ast, adt, zf)
    den = (den_p[0] + den_p[1]).reshape(NPAD, HEADS)[:N]

    aself = alpha_s + alpha_d
    exself = jnp.exp(jnp.maximum(aself, 0.2 * aself))
    denom = den + exself

    hp_pad = jnp.pad(hp, ((0, pad_n), (0, 0)))
    hps = tuple(hp_pad[:, i * C:(i + 1) * C] for i in range(HEADS))
    acc_p = _edge_phase_b(srcp, dstp, ex, hps, z32)
    acc = (acc_p[0] + acc_p[1]).transpose(1, 0, 2)[:N]
    numer = acc + exself[:, :, None] * hp.reshape(N, HEADS, C)
    out = numer / (denom[:, :, None] + 1e-16)
    return out.reshape(N, HID) + b


def kernel(x, edge_index, batch, emb_table, W1, a_src1, a_dst1, b1,
           W2, a_src2, a_dst2, b2, lin_w, lin_b):
    epad = E_PAD - E
    srcp = jnp.concatenate([edge_index[0].astype(jnp.int32),
                            jnp.zeros((epad,), jnp.int32)])
    dstp = jnp.concatenate([edge_index[1].astype(jnp.int32),
                            jnp.zeros((epad,), jnp.int32)])
    zf = jnp.zeros((NPAD * HEADS,), jnp.float32)
    z32 = jnp.zeros((NPAD, C), jnp.float32)

    npad = (-N) % (8 * _NW)
    x_pad = jnp.concatenate([x.astype(jnp.int32),
                             jnp.arange(npad, dtype=jnp.int32)])
    h = _emb_gather(emb_table, x_pad)[:N]
    h = jax.nn.relu(_gat_sc(h, srcp, dstp, W1, a_src1, a_dst1, b1, zf, z32))
    h = jax.nn.relu(_gat_sc(h, srcp, dstp, W2, a_src2, a_dst2, b2, zf, z32))

    pool_pad = 50176
    pt = (jnp.arange(NGRAPH)[:, None] == batch[None, :]).astype(jnp.float32)
    pt_pad = jnp.pad(pt, ((0, 0), (0, pool_pad - N)))
    h_pad = jnp.pad(h, ((0, pool_pad - N), (0, 0)))
    sums = _poolmm(pt_pad, h_pad)
    cnt = jnp.sum(pt, axis=1)
    mean = sums / jnp.maximum(cnt, 1.0)[:, None]
    lin_w_pad = jnp.pad(lin_w, ((0, 0), (0, HID - NCLASS)))
    out = _matmul(mean, lin_w_pad, block_m=NGRAPH)[:, :NCLASS]
    return out + lin_b
